# baseline (speedup 1.0000x reference)
import jax
import jax.numpy as jnp
from jax.experimental import pallas as pl

TH_HIGH = 0.5
TH_LOW = 0.3
V0 = 0.1
V1 = 0.2
IMG = 512.0
AB = 2048
KEY_MASK = 0x7FFF0000
KEY_ULP = 0x00010000


def _deinterleave(x, n):
    rows, width = x.shape
    run = 128 // n
    lane = jax.lax.broadcasted_iota(jnp.int32, (rows, 128), 1)
    idx = n * (lane % run) + lane // run
    pieces = [jnp.take_along_axis(x[:, p * 128:(p + 1) * 128], idx, axis=1)
              for p in range(width // 128)]
    return [jnp.concatenate([g[:, c * run:(c + 1) * run] for g in pieces],
                            axis=1) for c in range(n)]


def _phase_a(gt_ref, anc_ref, conf_ref, loc_ref,
             ce_ref, np_ref, pce_ref, pp1_ref, sl1_ref, perr_ref, e0_ref, e1_ref):
    ja = pl.program_id(0)

    @pl.when(ja == 0)
    def _():
        np_ref[...] = jnp.zeros_like(np_ref)
        pce_ref[...] = jnp.zeros_like(pce_ref)
        pp1_ref[...] = jnp.zeros_like(pp1_ref)
        sl1_ref[...] = jnp.zeros_like(sl1_ref)
        perr_ref[...] = jnp.zeros_like(perr_ref)
        e0_ref[...] = jnp.zeros_like(e0_ref)
        e1_ref[...] = jnp.zeros_like(e1_ref)

    ax1 = anc_ref[0:1, :]
    ay1 = anc_ref[1:2, :]
    ax2 = anc_ref[2:3, :]
    ay2 = anc_ref[3:4, :]
    area = anc_ref[4:5, :]
    cx = anc_ref[5:6, :]
    cy = anc_ref[6:7, :]
    w01 = anc_ref[7:8, :]
    h01 = anc_ref[8:9, :]
    logw = anc_ref[9:10, :]
    logh = anc_ref[10:11, :]
    w = anc_ref[11:12, :]
    h = anc_ref[12:13, :]

    gx1 = gt_ref[:, 0:1]
    gy1 = gt_ref[:, 1:2]
    gx2 = gt_ref[:, 2:3]
    gy2 = gt_ref[:, 3:4]
    garea = gt_ref[:, 4:5]
    gcx = gt_ref[:, 5:6]
    gcy = gt_ref[:, 6:7]
    loggw = gt_ref[:, 7:8]
    loggh = gt_ref[:, 8:9]

    wx = jnp.maximum(jnp.minimum(gx2, ax2) - jnp.maximum(gx1, ax1), 0.0)
    wy = jnp.maximum(jnp.minimum(gy2, ay2) - jnp.maximum(gy1, ay1), 0.0)
    inter = wx * wy
    iou = inter / (garea + area - inter)
    pos = iou >= TH_HIGH
    neg0 = iou <= TH_LOW
    posf = pos.astype(jnp.float32)

    l0, l1, l2, l3 = _deinterleave(loc_ref[...], 4)
    d0 = l0 - (gcx - cx) / w01
    d1 = l1 - (gcy - cy) / h01
    d2 = l2 - (loggw - logw) / V1
    d3 = l3 - (loggh - logh) / V1

    def sl1(d):
        ad = jnp.abs(d)
        return jnp.where(ad < 1.0, 0.5 * ad * ad, ad - 0.5)

    sl1_ref[...] += ((sl1(d0) + sl1(d1) + sl1(d2) + sl1(d3)) * posf).sum(
        axis=1, keepdims=True)

    whx = w * jnp.exp(l2 * V1)
    why = h * jnp.exp(l3 * V1)
    mnx = (cx + (l0 * V0) * w) - whx * 0.5
    mny = (cy + (l1 * V0) * h) - why * 0.5
    dx = (gx1 - mnx) * IMG
    dy = (gy1 - mny) * IMG
    perr_ref[...] += (jnp.sqrt(dx * dx + dy * dy) * posf).sum(axis=1, keepdims=True)
    e0_ref[...] += (jnp.abs(gx2 - (mnx + whx)) * posf).sum(axis=1, keepdims=True)
    e1_ref[...] += (jnp.abs(gy2 - (mny + why)) * posf).sum(axis=1, keepdims=True)

    c0, c1 = _deinterleave(conf_ref[...], 2)
    mx = jnp.maximum(c0, c1)
    lse = mx + jnp.log(1.0 + jnp.exp(-jnp.abs(c0 - c1)))
    ce0 = lse - c0
    ce1 = lse - c1
    pred1 = c1 > c0

    np_ref[...] += posf.sum(axis=1, keepdims=True)
    pce_ref[...] += (ce1 * posf).sum(axis=1, keepdims=True)
    pp1_ref[...] += (pred1.astype(jnp.float32) * posf).sum(axis=1, keepdims=True)

    cem = jnp.where(neg0, ce0, 0.0)
    ce_ref[...] = jnp.where(pred1, cem, -cem)[None]


def _phase_b(ce_ref, np_ref, pce_ref, pp1_ref, sl1_ref, perr_ref, e0_ref, e1_ref,
             out_ref):
    nchunks = ce_ref.shape[0]
    rows = ce_ref.shape[1]
    a_tot = nchunks * ce_ref.shape[2]
    npos = np_ref[...]
    k = jnp.clip(3.0 * npos, 10.0, float(a_tot - 1))

    def count_gt(thr):
        def ch(c, s):
            bits = jax.lax.bitcast_convert_type(ce_ref[c], jnp.int32) & KEY_MASK
            return s + (bits > thr).astype(jnp.float32).sum(axis=1, keepdims=True)
        return jax.lax.fori_loop(0, nchunks, ch,
                                 jnp.zeros((rows, 1), jnp.float32))

    def bit_step(i, m):
        cand = m | (jnp.int32(1) << (30 - i))
        return jnp.where(count_gt(cand) >= k, cand, m)

    m = jax.lax.fori_loop(0, 15, bit_step, jnp.zeros((rows, 1), jnp.int32))
    kstar = m + KEY_ULP
    tval = jax.lax.bitcast_convert_type(kstar, jnp.float32)

    def fin(c, carry):
        cg, sce, sp0g, ct, sp0t = carry
        blk = ce_ref[c]
        bi = jax.lax.bitcast_convert_type(blk, jnp.int32)
        bits = bi & KEY_MASK
        pred0 = (bi < 0).astype(jnp.float32)
        g = (bits > kstar).astype(jnp.float32)
        t = (bits == kstar).astype(jnp.float32)
        ce = jnp.abs(blk)
        return (cg + g.sum(axis=1, keepdims=True),
                sce + (ce * g).sum(axis=1, keepdims=True),
                sp0g + (pred0 * g).sum(axis=1, keepdims=True),
                ct + t.sum(axis=1, keepdims=True),
                sp0t + (pred0 * t).sum(axis=1, keepdims=True))

    z = jnp.zeros((rows, 1), jnp.float32)
    cg, sce, sp0g, ct, sp0t = jax.lax.fori_loop(0, nchunks, fin,
                                                (z, z, z, z, z))
    rem = k - cg
    s_negce = jnp.sum(sce + tval * rem)
    s_negp0 = jnp.sum(sp0g + rem * sp0t / jnp.maximum(ct, 1.0))

    n = jnp.sum(npos)
    nf = jnp.maximum(n, 1.0)
    s_k = jnp.sum(k)
    loss_loc = jnp.sum(sl1_ref[...]) / (nf * 4.0)
    loss_cls = (jnp.sum(pce_ref[...]) + s_negce * (1.0 / 3.0)) / jnp.maximum(
        n + s_k * (1.0 / 3.0), 1e-12)
    pos_acc = jnp.sum(pp1_ref[...]) / nf
    neg_acc = s_negp0 / jnp.maximum(s_k, 1.0)
    pos_err = jnp.sum(perr_ref[...]) / nf
    se0 = jnp.sum(e0_ref[...]) / nf * IMG
    se1 = jnp.sum(e1_ref[...]) / nf * IMG

    vals = [loss_loc, loss_cls, pos_acc, neg_acc, pos_err, se0, se1, n]
    out_ref[...] = jnp.concatenate(
        [jnp.broadcast_to(v, (1, 128)) for v in vals], axis=0)


def kernel(loc_pred, conf_pred, gt_boxes, anchors):
    b, a = loc_pred.shape[0], loc_pred.shape[1]
    na = a // AB

    conf_t = conf_pred.reshape(b, 2 * a)
    loc_t = loc_pred.reshape(b, 4 * a)

    cx, cy, w, h = anchors[:, 0], anchors[:, 1], anchors[:, 2], anchors[:, 3]
    ax1 = cx - w / 2.0
    ay1 = cy - h / 2.0
    ax2 = cx + w / 2.0
    ay2 = cy + h / 2.0
    area = (ax2 - ax1) * (ay2 - ay1)
    zero = jnp.zeros_like(w)
    anc = jnp.stack([ax1, ay1, ax2, ay2, area, cx, cy, V0 * w, V0 * h,
                     jnp.log(w), jnp.log(h), w, h, zero, zero, zero], axis=0)

    gx1, gy1, gx2, gy2 = (gt_boxes[:, 0], gt_boxes[:, 1],
                          gt_boxes[:, 2], gt_boxes[:, 3])
    gtc = jnp.stack([gx1, gy1, gx2, gy2, (gx2 - gx1) * (gy2 - gy1),
                     (gx1 + gx2) / 2.0, (gy1 + gy2) / 2.0,
                     jnp.log(gx2 - gx1), jnp.log(gy2 - gy1)], axis=1)
    gtc = jnp.pad(gtc, ((0, 0), (0, 128 - gtc.shape[1])))

    row_spec = pl.BlockSpec((b, 1), lambda j: (0, 0))
    f32 = jnp.float32
    ce, s_np, s_pce, s_pp1, s_sl1, s_perr, s_e0, s_e1 = pl.pallas_call(
        _phase_a,
        grid=(na,),
        in_specs=[pl.BlockSpec((b, 128), lambda j: (0, 0)),
                  pl.BlockSpec((16, AB), lambda j: (0, j)),
                  pl.BlockSpec((b, 2 * AB), lambda j: (0, j)),
                  pl.BlockSpec((b, 4 * AB), lambda j: (0, j))],
        out_specs=[pl.BlockSpec((1, b, AB), lambda j: (j, 0, 0)),
                   row_spec, row_spec, row_spec, row_spec, row_spec,
                   row_spec, row_spec],
        out_shape=[jax.ShapeDtypeStruct((na, b, AB), f32)] +
                  [jax.ShapeDtypeStruct((b, 1), f32)] * 7,
    )(gtc, anc, conf_t, loc_t)

    full_row = pl.BlockSpec((b, 1), lambda: (0, 0))
    out = pl.pallas_call(
        _phase_b,
        in_specs=[pl.BlockSpec((na, b, AB), lambda: (0, 0, 0)),
                  full_row, full_row, full_row, full_row, full_row,
                  full_row, full_row],
        out_specs=pl.BlockSpec((8, 128), lambda: (0, 0)),
        out_shape=jax.ShapeDtypeStruct((8, 128), f32),
    )(ce, s_np, s_pce, s_pp1, s_sl1, s_perr, s_e0, s_e1)

    return (out[0, 0], out[1, 0], out[2, 0], out[3, 0], out[4, 0],
            jnp.stack([out[5, 0], out[6, 0]]), out[7, 0].astype(jnp.int32))


# speedup vs baseline: 1.5380x; 1.5380x over previous
import jax
import jax.numpy as jnp
from jax.experimental import pallas as pl
from jax.experimental.pallas import tpu as pltpu

TH_HIGH = 0.5
TH_LOW = 0.3
V0 = 0.1
V1 = 0.2
IMG = 512.0
AB = 2048
KEY_MASK = 0x7FFF0000
KEY_ULP = 0x00010000


def _mine_and_finalize(ce_s, np_s, pce_s, pp1_s, sl1_s, perr_s, e0_s, e1_s,
                       out_ref):
    nchunks = ce_s.shape[0]
    rows = ce_s.shape[1]
    a_tot = nchunks * ce_s.shape[2]
    npos = np_s[...]
    k = jnp.clip(3.0 * npos, 10.0, float(a_tot - 1))

    def count_gt(thr):
        def ch(c, s):
            bits = jax.lax.bitcast_convert_type(ce_s[c], jnp.int32) & KEY_MASK
            return s + (bits > thr).astype(jnp.float32).sum(axis=1, keepdims=True)
        return jax.lax.fori_loop(0, nchunks, ch,
                                 jnp.zeros((rows, 1), jnp.float32))

    def bit_step(i, m):
        cand = m | (jnp.int32(1) << (30 - i))
        return jnp.where(count_gt(cand) >= k, cand, m)

    m = jax.lax.fori_loop(0, 15, bit_step, jnp.zeros((rows, 1), jnp.int32))
    kstar = m + KEY_ULP
    tval = jax.lax.bitcast_convert_type(kstar, jnp.float32)

    def fin(c, carry):
        cg, sce, sp0g, ct, sp0t = carry
        blk = ce_s[c]
        bi = jax.lax.bitcast_convert_type(blk, jnp.int32)
        bits = bi & KEY_MASK
        pred0 = (bi < 0).astype(jnp.float32)
        g = (bits > kstar).astype(jnp.float32)
        t = (bits == kstar).astype(jnp.float32)
        ce = jnp.abs(blk)
        return (cg + g.sum(axis=1, keepdims=True),
                sce + (ce * g).sum(axis=1, keepdims=True),
                sp0g + (pred0 * g).sum(axis=1, keepdims=True),
                ct + t.sum(axis=1, keepdims=True),
                sp0t + (pred0 * t).sum(axis=1, keepdims=True))

    z = jnp.zeros((rows, 1), jnp.float32)
    cg, sce, sp0g, ct, sp0t = jax.lax.fori_loop(0, nchunks, fin,
                                                (z, z, z, z, z))
    rem = k - cg
    s_negce = jnp.sum(sce + tval * rem)
    s_negp0 = jnp.sum(sp0g + rem * sp0t / jnp.maximum(ct, 1.0))

    n = jnp.sum(npos)
    nf = jnp.maximum(n, 1.0)
    s_k = jnp.sum(k)
    loss_loc = jnp.sum(sl1_s[...]) / (nf * 4.0)
    loss_cls = (jnp.sum(pce_s[...]) + s_negce * (1.0 / 3.0)) / jnp.maximum(
        n + s_k * (1.0 / 3.0), 1e-12)
    pos_acc = jnp.sum(pp1_s[...]) / nf
    neg_acc = s_negp0 / jnp.maximum(s_k, 1.0)
    pos_err = jnp.sum(perr_s[...]) / nf
    se0 = jnp.sum(e0_s[...]) / nf * IMG
    se1 = jnp.sum(e1_s[...]) / nf * IMG

    vals = [loss_loc, loss_cls, pos_acc, neg_acc, pos_err, se0, se1, n]
    out_ref[...] = jnp.concatenate(
        [jnp.broadcast_to(v, (1, 128)) for v in vals], axis=0)


def _fused(gt_ref, anc_ref, conf_ref, loc_ref, out_ref,
           ce_s, np_s, pce_s, pp1_s, sl1_s, perr_s, e0_s, e1_s):
    ja = pl.program_id(0)

    @pl.when(ja == 0)
    def _():
        np_s[...] = jnp.zeros_like(np_s)
        pce_s[...] = jnp.zeros_like(pce_s)
        pp1_s[...] = jnp.zeros_like(pp1_s)
        sl1_s[...] = jnp.zeros_like(sl1_s)
        perr_s[...] = jnp.zeros_like(perr_s)
        e0_s[...] = jnp.zeros_like(e0_s)
        e1_s[...] = jnp.zeros_like(e1_s)

    ax1 = anc_ref[0:1, :]
    ay1 = anc_ref[1:2, :]
    ax2 = anc_ref[2:3, :]
    ay2 = anc_ref[3:4, :]
    area = anc_ref[4:5, :]
    cx = anc_ref[5:6, :]
    cy = anc_ref[6:7, :]
    rw01 = anc_ref[7:8, :]
    rh01 = anc_ref[8:9, :]
    logw = anc_ref[9:10, :]
    logh = anc_ref[10:11, :]
    w = anc_ref[11:12, :]
    h = anc_ref[12:13, :]

    gx1 = gt_ref[:, 0:1]
    gy1 = gt_ref[:, 1:2]
    gx2 = gt_ref[:, 2:3]
    gy2 = gt_ref[:, 3:4]
    garea = gt_ref[:, 4:5]
    gcx = gt_ref[:, 5:6]
    gcy = gt_ref[:, 6:7]
    loggw = gt_ref[:, 7:8]
    loggh = gt_ref[:, 8:9]

    wx = jnp.maximum(jnp.minimum(gx2, ax2) - jnp.maximum(gx1, ax1), 0.0)
    wy = jnp.maximum(jnp.minimum(gy2, ay2) - jnp.maximum(gy1, ay1), 0.0)
    inter = wx * wy
    iou = inter / (garea + area - inter)
    pos = iou >= TH_HIGH
    neg0 = iou <= TH_LOW
    posf = pos.astype(jnp.float32)

    l0 = loc_ref[0]
    l1 = loc_ref[1]
    l2 = loc_ref[2]
    l3 = loc_ref[3]
    d0 = l0 - (gcx - cx) * rw01
    d1 = l1 - (gcy - cy) * rh01
    d2 = l2 - (loggw - logw) * (1.0 / V1)
    d3 = l3 - (loggh - logh) * (1.0 / V1)

    def sl1(d):
        ad = jnp.abs(d)
        return jnp.where(ad < 1.0, 0.5 * ad * ad, ad - 0.5)

    sl1_s[...] += ((sl1(d0) + sl1(d1) + sl1(d2) + sl1(d3)) * posf).sum(
        axis=1, keepdims=True)

    whx = w * jnp.exp(l2 * V1)
    why = h * jnp.exp(l3 * V1)
    mnx = (cx + (l0 * V0) * w) - whx * 0.5
    mny = (cy + (l1 * V0) * h) - why * 0.5
    dx = (gx1 - mnx) * IMG
    dy = (gy1 - mny) * IMG
    perr_s[...] += (jnp.sqrt(dx * dx + dy * dy) * posf).sum(axis=1, keepdims=True)
    e0_s[...] += (jnp.abs(gx2 - (mnx + whx)) * posf).sum(axis=1, keepdims=True)
    e1_s[...] += (jnp.abs(gy2 - (mny + why)) * posf).sum(axis=1, keepdims=True)

    c0 = conf_ref[0]
    c1 = conf_ref[1]
    mx = jnp.maximum(c0, c1)
    lse = mx + jnp.log(1.0 + jnp.exp(-jnp.abs(c0 - c1)))
    ce0 = lse - c0
    ce1 = lse - c1
    pred1 = c1 > c0

    np_s[...] += posf.sum(axis=1, keepdims=True)
    pce_s[...] += (ce1 * posf).sum(axis=1, keepdims=True)
    pp1_s[...] += (pred1.astype(jnp.float32) * posf).sum(axis=1, keepdims=True)

    cem = jnp.where(neg0, ce0, 0.0)
    ce_s[ja] = jnp.where(pred1, cem, -cem)

    @pl.when(ja == pl.num_programs(0) - 1)
    def _():
        _mine_and_finalize(ce_s, np_s, pce_s, pp1_s, sl1_s, perr_s, e0_s, e1_s,
                           out_ref)


def kernel(loc_pred, conf_pred, gt_boxes, anchors):
    b, a = loc_pred.shape[0], loc_pred.shape[1]
    na = a // AB

    conf_t = conf_pred.transpose(2, 0, 1)
    loc_t = loc_pred.transpose(2, 0, 1)

    cx, cy, w, h = anchors[:, 0], anchors[:, 1], anchors[:, 2], anchors[:, 3]
    ax1 = cx - w / 2.0
    ay1 = cy - h / 2.0
    ax2 = cx + w / 2.0
    ay2 = cy + h / 2.0
    area = (ax2 - ax1) * (ay2 - ay1)
    zero = jnp.zeros_like(w)
    anc = jnp.stack([ax1, ay1, ax2, ay2, area, cx, cy,
                     1.0 / (V0 * w), 1.0 / (V0 * h),
                     jnp.log(w), jnp.log(h), w, h, zero, zero, zero], axis=0)

    gx1, gy1, gx2, gy2 = (gt_boxes[:, 0], gt_boxes[:, 1],
                          gt_boxes[:, 2], gt_boxes[:, 3])
    gtc = jnp.stack([gx1, gy1, gx2, gy2, (gx2 - gx1) * (gy2 - gy1),
                     (gx1 + gx2) / 2.0, (gy1 + gy2) / 2.0,
                     jnp.log(gx2 - gx1), jnp.log(gy2 - gy1)], axis=1)
    gtc = jnp.pad(gtc, ((0, 0), (0, 128 - gtc.shape[1])))

    f32 = jnp.float32
    out = pl.pallas_call(
        _fused,
        grid=(na,),
        in_specs=[pl.BlockSpec((b, 128), lambda j: (0, 0)),
                  pl.BlockSpec((16, AB), lambda j: (0, j)),
                  pl.BlockSpec((2, b, AB), lambda j: (0, 0, j)),
                  pl.BlockSpec((4, b, AB), lambda j: (0, 0, j))],
        out_specs=pl.BlockSpec((8, 128), lambda j: (0, 0)),
        out_shape=jax.ShapeDtypeStruct((8, 128), f32),
        scratch_shapes=[pltpu.VMEM((na, b, AB), f32)] +
                       [pltpu.VMEM((b, 1), f32)] * 7,
    )(gtc, anc, conf_t, loc_t)

    return (out[0, 0], out[1, 0], out[2, 0], out[3, 0], out[4, 0],
            jnp.stack([out[5, 0], out[6, 0]]), out[7, 0].astype(jnp.int32))


# speedup vs baseline: 1.6362x; 1.0639x over previous
import jax
import jax.numpy as jnp
from jax.experimental import pallas as pl
from jax.experimental.pallas import tpu as pltpu

TH_HIGH = 0.5
TH_LOW = 0.3
V0 = 0.1
V1 = 0.2
IMG = 512.0
AB = 2048
KEY_MASK = 0x7FFF0000
KEY_ULP = 0x00010000


def _mine_and_finalize(ce_s, pk_s, np_s, pce_s, pp1_s, sl1_s, perr_s, e0_s,
                       e1_s, out_ref):
    nchunks = ce_s.shape[0]
    rows = ce_s.shape[1]
    a_tot = nchunks * ce_s.shape[2]
    npos = np_s[...]
    k = jnp.clip(3.0 * npos, 10.0, float(a_tot - 1))

    def packer(cc, t):
        hi = (jax.lax.bitcast_convert_type(ce_s[2 * cc], jnp.int32) >> 16) & 0x7FFF
        lo = (jax.lax.bitcast_convert_type(ce_s[2 * cc + 1], jnp.int32) >> 16) & 0x7FFF
        pk_s[cc] = (hi << 16) | lo
        return t

    jax.lax.fori_loop(0, nchunks // 2, packer, 0)

    def count_gt(thr):
        def ch(c, s):
            wv = pk_s[c]
            cnt = ((wv >> 16) > thr).astype(jnp.float32) + \
                  ((wv & 0xFFFF) > thr).astype(jnp.float32)
            return s + cnt.sum(axis=1, keepdims=True)
        return jax.lax.fori_loop(0, nchunks // 2, ch,
                                 jnp.zeros((rows, 1), jnp.float32))

    def bit_step(i, m):
        cand = m | (jnp.int32(1) << (14 - i))
        return jnp.where(count_gt(cand) >= k, cand, m)

    m = jax.lax.fori_loop(0, 15, bit_step, jnp.zeros((rows, 1), jnp.int32))
    kstar = (m + 1) << 16
    tval = jax.lax.bitcast_convert_type(kstar, jnp.float32)

    def fin(c, carry):
        cg, sce, sp0g, ct, sp0t = carry
        blk = ce_s[c]
        bi = jax.lax.bitcast_convert_type(blk, jnp.int32)
        bits = bi & KEY_MASK
        pred0 = (bi < 0).astype(jnp.float32)
        g = (bits > kstar).astype(jnp.float32)
        t = (bits == kstar).astype(jnp.float32)
        ce = jnp.abs(blk)
        return (cg + g.sum(axis=1, keepdims=True),
                sce + (ce * g).sum(axis=1, keepdims=True),
                sp0g + (pred0 * g).sum(axis=1, keepdims=True),
                ct + t.sum(axis=1, keepdims=True),
                sp0t + (pred0 * t).sum(axis=1, keepdims=True))

    z = jnp.zeros((rows, 1), jnp.float32)
    cg, sce, sp0g, ct, sp0t = jax.lax.fori_loop(0, nchunks, fin,
                                                (z, z, z, z, z))
    rem = k - cg
    s_negce = jnp.sum(sce + tval * rem)
    s_negp0 = jnp.sum(sp0g + rem * sp0t / jnp.maximum(ct, 1.0))

    n = jnp.sum(npos)
    nf = jnp.maximum(n, 1.0)
    s_k = jnp.sum(k)
    loss_loc = jnp.sum(sl1_s[...]) / (nf * 4.0)
    loss_cls = (jnp.sum(pce_s[...]) + s_negce * (1.0 / 3.0)) / jnp.maximum(
        n + s_k * (1.0 / 3.0), 1e-12)
    pos_acc = jnp.sum(pp1_s[...]) / nf
    neg_acc = s_negp0 / jnp.maximum(s_k, 1.0)
    pos_err = jnp.sum(perr_s[...]) / nf
    se0 = jnp.sum(e0_s[...]) / nf * IMG
    se1 = jnp.sum(e1_s[...]) / nf * IMG

    vals = [loss_loc, loss_cls, pos_acc, neg_acc, pos_err, se0, se1, n]
    out_ref[...] = jnp.concatenate(
        [jnp.broadcast_to(v, (1, 128)) for v in vals], axis=0)


def _fused(gt_ref, anc_ref, conf_ref, loc_ref, out_ref,
           ce_s, pk_s, np_s, pce_s, pp1_s, sl1_s, perr_s, e0_s, e1_s):
    ja = pl.program_id(0)

    @pl.when(ja == 0)
    def _():
        np_s[...] = jnp.zeros_like(np_s)
        pce_s[...] = jnp.zeros_like(pce_s)
        pp1_s[...] = jnp.zeros_like(pp1_s)
        sl1_s[...] = jnp.zeros_like(sl1_s)
        perr_s[...] = jnp.zeros_like(perr_s)
        e0_s[...] = jnp.zeros_like(e0_s)
        e1_s[...] = jnp.zeros_like(e1_s)

    ax1 = anc_ref[0:1, :]
    ay1 = anc_ref[1:2, :]
    ax2 = anc_ref[2:3, :]
    ay2 = anc_ref[3:4, :]
    area = anc_ref[4:5, :]
    cx = anc_ref[5:6, :]
    cy = anc_ref[6:7, :]
    rw01 = anc_ref[7:8, :]
    rh01 = anc_ref[8:9, :]
    logw = anc_ref[9:10, :]
    logh = anc_ref[10:11, :]
    w = anc_ref[11:12, :]
    h = anc_ref[12:13, :]

    gx1 = gt_ref[:, 0:1]
    gy1 = gt_ref[:, 1:2]
    gx2 = gt_ref[:, 2:3]
    gy2 = gt_ref[:, 3:4]
    garea = gt_ref[:, 4:5]
    gcx = gt_ref[:, 5:6]
    gcy = gt_ref[:, 6:7]
    loggw = gt_ref[:, 7:8]
    loggh = gt_ref[:, 8:9]

    wx = jnp.maximum(jnp.minimum(gx2, ax2) - jnp.maximum(gx1, ax1), 0.0)
    wy = jnp.maximum(jnp.minimum(gy2, ay2) - jnp.maximum(gy1, ay1), 0.0)
    inter = wx * wy
    iou = inter / (garea + area - inter)
    pos = iou >= TH_HIGH
    neg0 = iou <= TH_LOW
    posf = pos.astype(jnp.float32)

    l0 = loc_ref[0]
    l1 = loc_ref[1]
    l2 = loc_ref[2]
    l3 = loc_ref[3]
    d0 = l0 - (gcx - cx) * rw01
    d1 = l1 - (gcy - cy) * rh01
    d2 = l2 - (loggw - logw) * (1.0 / V1)
    d3 = l3 - (loggh - logh) * (1.0 / V1)

    def sl1(d):
        ad = jnp.abs(d)
        return jnp.where(ad < 1.0, 0.5 * ad * ad, ad - 0.5)

    sl1_s[...] += ((sl1(d0) + sl1(d1) + sl1(d2) + sl1(d3)) * posf).sum(
        axis=1, keepdims=True)

    whx = w * jnp.exp(l2 * V1)
    why = h * jnp.exp(l3 * V1)
    mnx = (cx + (l0 * V0) * w) - whx * 0.5
    mny = (cy + (l1 * V0) * h) - why * 0.5
    dx = (gx1 - mnx) * IMG
    dy = (gy1 - mny) * IMG
    perr_s[...] += (jnp.sqrt(dx * dx + dy * dy) * posf).sum(axis=1, keepdims=True)
    e0_s[...] += (jnp.abs(gx2 - (mnx + whx)) * posf).sum(axis=1, keepdims=True)
    e1_s[...] += (jnp.abs(gy2 - (mny + why)) * posf).sum(axis=1, keepdims=True)

    c0 = conf_ref[0]
    c1 = conf_ref[1]
    mx = jnp.maximum(c0, c1)
    lse = mx + jnp.log(1.0 + jnp.exp(-jnp.abs(c0 - c1)))
    ce0 = lse - c0
    ce1 = lse - c1
    pred1 = c1 > c0

    np_s[...] += posf.sum(axis=1, keepdims=True)
    pce_s[...] += (ce1 * posf).sum(axis=1, keepdims=True)
    pp1_s[...] += (pred1.astype(jnp.float32) * posf).sum(axis=1, keepdims=True)

    cem = jnp.where(neg0, ce0, 0.0)
    ce_s[ja] = jnp.where(pred1, cem, -cem)

    @pl.when(ja == pl.num_programs(0) - 1)
    def _():
        _mine_and_finalize(ce_s, pk_s, np_s, pce_s, pp1_s, sl1_s, perr_s, e0_s,
                           e1_s, out_ref)


def kernel(loc_pred, conf_pred, gt_boxes, anchors):
    b, a = loc_pred.shape[0], loc_pred.shape[1]
    na = a // AB

    conf_t = conf_pred.transpose(2, 0, 1)
    loc_t = loc_pred.transpose(2, 0, 1)

    cx, cy, w, h = anchors[:, 0], anchors[:, 1], anchors[:, 2], anchors[:, 3]
    ax1 = cx - w / 2.0
    ay1 = cy - h / 2.0
    ax2 = cx + w / 2.0
    ay2 = cy + h / 2.0
    area = (ax2 - ax1) * (ay2 - ay1)
    zero = jnp.zeros_like(w)
    anc = jnp.stack([ax1, ay1, ax2, ay2, area, cx, cy,
                     1.0 / (V0 * w), 1.0 / (V0 * h),
                     jnp.log(w), jnp.log(h), w, h, zero, zero, zero], axis=0)

    gx1, gy1, gx2, gy2 = (gt_boxes[:, 0], gt_boxes[:, 1],
                          gt_boxes[:, 2], gt_boxes[:, 3])
    gtc = jnp.stack([gx1, gy1, gx2, gy2, (gx2 - gx1) * (gy2 - gy1),
                     (gx1 + gx2) / 2.0, (gy1 + gy2) / 2.0,
                     jnp.log(gx2 - gx1), jnp.log(gy2 - gy1)], axis=1)
    gtc = jnp.pad(gtc, ((0, 0), (0, 128 - gtc.shape[1])))

    f32 = jnp.float32
    out = pl.pallas_call(
        _fused,
        grid=(na,),
        in_specs=[pl.BlockSpec((b, 128), lambda j: (0, 0)),
                  pl.BlockSpec((16, AB), lambda j: (0, j)),
                  pl.BlockSpec((2, b, AB), lambda j: (0, 0, j)),
                  pl.BlockSpec((4, b, AB), lambda j: (0, 0, j))],
        out_specs=pl.BlockSpec((8, 128), lambda j: (0, 0)),
        out_shape=jax.ShapeDtypeStruct((8, 128), f32),
        scratch_shapes=[pltpu.VMEM((na, b, AB), f32),
                        pltpu.VMEM((na // 2, b, AB), jnp.int32)] +
                       [pltpu.VMEM((b, 1), f32)] * 7,
    )(gtc, anc, conf_t, loc_t)

    return (out[0, 0], out[1, 0], out[2, 0], out[3, 0], out[4, 0],
            jnp.stack([out[5, 0], out[6, 0]]), out[7, 0].astype(jnp.int32))
